# Initial kernel scaffold; baseline (speedup 1.0000x reference)
import jax
import jax.numpy as jnp
from jax.experimental import pallas as pl


def kernel(inputs, kernel_points, W):
    raise NotImplementedError("write your pallas kernel here")


# speedup vs baseline: 6.8755x; 6.8755x over previous
import jax
import jax.numpy as jnp
from jax.experimental import pallas as pl
from jax.experimental.pallas import tpu as pltpu

_B, _N, _CIN = 4, 4096, 67
_S, _K, _F = 16, 15, 64
_KP_EXTEND = 0.1
_ALPHA = 0.3
_NQ = 256
_R2MAX = (2.0 * _KP_EXTEND) ** 2
_BIG = 1e30


def _kpconv_block(xyzT_ref, qxyz_ref, feats_ref, kp_ref, kp2_ref, w_ref, out_ref):
    xyzT = xyzT_ref[0]
    q = qxyz_ref[0]
    feats = feats_ref[0]

    x0, x1, x2 = xyzT[0:1, :], xyzT[1:2, :], xyzT[2:3, :]
    q0, q1, q2 = q[:, 0:1], q[:, 1:2], q[:, 2:3]

    dx = q0 - x0
    dy = q1 - x1
    dz = q2 - x2
    d2 = (dx * dx + dy * dy) + dz * dz

    work = jnp.where(d2 < _R2MAX, d2, _BIG)
    t = jnp.min(work, axis=1, keepdims=True)
    for _ in range(_S - 1):
        work = jnp.where(work <= t, _BIG, work)
        t = jnp.min(work, axis=1, keepdims=True)
    maskf = jnp.where((d2 < _R2MAX) & (d2 <= t), 1.0, 0.0)

    out = jnp.zeros((_NQ, _F), jnp.float32)
    for k in range(_K):
        ax, ay, az = kp_ref[k, 0], kp_ref[k, 1], kp_ref[k, 2]
        projm = ax * x0 + ay * x1 + az * x2
        projq = ax * q0 + ay * q1 + az * q2
        dk2 = d2 - 2.0 * projm + (2.0 * projq + kp2_ref[k])
        dist = jnp.sqrt(jnp.maximum(dk2, 0.0) + 1e-12)
        w = jnp.maximum(0.0, 1.0 - dist * (1.0 / _KP_EXTEND)) * maskf
        h = jax.lax.dot_general(w, feats, (((1,), (0,)), ((), ())),
                                preferred_element_type=jnp.float32)
        out = out + jax.lax.dot_general(h, w_ref[k], (((1,), (0,)), ((), ())),
                                        preferred_element_type=jnp.float32)
    out_ref[0] = jnp.where(out > 0, out, _ALPHA * out)


def kernel(inputs, kernel_points, W):
    xyz = inputs[..., :3]
    xyzT = jnp.transpose(xyz, (0, 2, 1))
    kp2 = jnp.sum(kernel_points * kernel_points, axis=1)

    return pl.pallas_call(
        _kpconv_block,
        grid=(_B, _N // _NQ),
        in_specs=[
            pl.BlockSpec((1, 3, _N), lambda b, q: (b, 0, 0)),
            pl.BlockSpec((1, _NQ, 3), lambda b, q: (b, q, 0)),
            pl.BlockSpec((1, _N, _CIN), lambda b, q: (b, 0, 0)),
            pl.BlockSpec(memory_space=pltpu.SMEM),
            pl.BlockSpec(memory_space=pltpu.SMEM),
            pl.BlockSpec((_K, _CIN, _F), lambda b, q: (0, 0, 0)),
        ],
        out_specs=pl.BlockSpec((1, _NQ, _F), lambda b, q: (b, q, 0)),
        out_shape=jax.ShapeDtypeStruct((_B, _N, _F), jnp.float32),
    )(xyzT, xyz, inputs, kernel_points, kp2, W)
